# baseline (speedup 1.0000x reference)
import functools
import math

import jax
import jax.numpy as jnp
from jax import lax
from jax.experimental import pallas as pl
from jax.experimental.pallas import tpu as pltpu
from jax.experimental.pallas import tpu_sc as plsc

_CUTOFF = 4.0
_NUM_BASIS = 16

_NC = 2
_NS = 16
_NW = _NC * _NS

_C = 2000

_B = 4096


def _sc_edge_diff(pos_flat, ei_flat, n, e_pad):
    e = ei_flat.shape[0] // 2
    assert e % _NW == 0
    ew = e // _NW
    assert ew % _C == 0

    mesh = plsc.VectorSubcoreMesh(core_axis_name="c", subcore_axis_name="s")
    out_type = (
        jax.ShapeDtypeStruct((e_pad,), jnp.float32),
        jax.ShapeDtypeStruct((e_pad,), jnp.float32),
        jax.ShapeDtypeStruct((e_pad,), jnp.float32),
    )
    scratch = [
        pltpu.VMEM((n,), jnp.float32),
        pltpu.VMEM((_C,), jnp.int32),
        pltpu.VMEM((_C,), jnp.int32),
        pltpu.VMEM((_C,), jnp.float32),
    ]

    @functools.partial(
        pl.kernel, out_type=out_type, mesh=mesh, scratch_types=scratch,
        compiler_params=pltpu.CompilerParams(needs_layout_passes=False))
    def sc_kernel(pos_hbm, ei_hbm, d0_hbm, d1_hbm, d2_hbm,
                  plane, sidx, didx, dbuf):
        wid = lax.axis_index("s") * _NC + lax.axis_index("c")

        for pofs, d_hbm in ((0, d0_hbm), (n, d1_hbm), (2 * n, d2_hbm)):
            pltpu.sync_copy(pos_hbm.at[pl.ds(pofs, n)], plane)

            def chunk(kk, _):
                cb = wid * ew + kk * _C
                pltpu.sync_copy(ei_hbm.at[pl.ds(cb, _C)], sidx)
                pltpu.sync_copy(ei_hbm.at[pl.ds(e + cb, _C)], didx)

                def step(i, _):
                    sl = pl.ds(i * 16, 16)
                    a = plsc.load_gather(plane, [didx[sl]])
                    b = plsc.load_gather(plane, [sidx[sl]])
                    dbuf[sl] = a - b
                    return 0

                lax.fori_loop(0, _C // 16, step, 0)
                pltpu.sync_copy(dbuf, d_hbm.at[pl.ds(cb, _C)])
                return 0

            lax.fori_loop(0, ew // _C, chunk, 0)

    return sc_kernel(pos_flat, ei_flat)


def _tc_node_embed(x, emb2t):
    n = x.shape[0]
    b = 3200
    assert n % b == 0
    nb = n // b
    xr = x.reshape(nb, 1, b)
    s = emb2t.shape[1]

    def body(x_ref, e_ref, hx_ref, hz_ref):
        xv = x_ref[...].reshape(1, b)
        sp = lax.broadcasted_iota(jnp.int32, (s, 1), 0)
        onehot = (xv == sp).astype(jnp.float32)
        t = lax.dot_general(e_ref[...], onehot, (((1,), (0,)), ((), ())),
                            preferred_element_type=jnp.float32)
        hx_ref[...] = t[:8, :]
        hz_ref[...] = t[8:, :]

    return pl.pallas_call(
        body,
        grid=(nb,),
        in_specs=[pl.BlockSpec((1, 1, b), lambda i: (i, 0, 0)),
                  pl.BlockSpec((16, s), lambda i: (0, 0))],
        out_specs=[pl.BlockSpec((8, b), lambda i: (0, i)),
                   pl.BlockSpec((8, b), lambda i: (0, i))],
        out_shape=[jax.ShapeDtypeStruct((8, n), jnp.float32),
                   jax.ShapeDtypeStruct((8, n), jnp.float32)],
    )(xr, emb2t)


_PI_A = 3.140625
_PI_B = 0.0009676535897932795
_S0 = 9.9999999372e-01
_S1 = -1.6666655189e-01
_S2 = 8.3329909945e-03
_S3 = -1.9805000098e-04
_S4 = 2.5966513689e-06
_K0 = 1.0
_K1 = -4.9999999725e-01
_K2 = 4.1666418880e-02
_K3 = -1.3887316255e-03
_K4 = 2.4433157103e-05


def _fast_sincos(x):
    kf = jnp.floor(x * (1.0 / math.pi) + 0.5)
    y = (x - kf * _PI_A) - kf * _PI_B
    y2 = y * y
    ps = y * (_S0 + y2 * (_S1 + y2 * (_S2 + y2 * (_S3 + y2 * _S4))))
    pc = _K0 + y2 * (_K1 + y2 * (_K2 + y2 * (_K3 + y2 * _K4)))
    sbit = jnp.left_shift(jnp.bitwise_and(kf.astype(jnp.int32), 1), 31)
    sb = lax.bitcast_convert_type(ps, jnp.int32) ^ sbit
    cb = lax.bitcast_convert_type(pc, jnp.int32) ^ sbit
    return (lax.bitcast_convert_type(sb, jnp.float32),
            lax.bitcast_convert_type(cb, jnp.float32))


def _tc_bessel(d0, d1, d2):
    e_pad = d0.shape[0]
    assert e_pad % _B == 0
    nb = e_pad // _B
    dr = [d.reshape(nb, 8, 512) for d in (d0, d1, d2)]

    def body(xr, yr, zr, h_ref, a_ref):
        dx = xr[...].reshape(8, 512)
        dy = yr[...].reshape(8, 512)
        dz = zr[...].reshape(8, 512)
        r2 = dx * dx + dy * dy + dz * dz
        inv_r = lax.rsqrt(r2)
        g = math.sqrt(2.0 / _CUTOFF) * inv_r
        theta = r2 * inv_r * (math.pi / _CUTOFF)
        s1, c1 = _fast_sincos(theta)
        c2 = c1 + c1
        h1 = s1 * g
        h2 = c2 * h1
        a_ref[0, 0] = dx
        a_ref[1, 0] = dy
        a_ref[2, 0] = dz
        h_ref[0, 0] = h1
        h_ref[1, 0] = h2
        hm2, hm1 = h1, h2
        for nn in range(2, _NUM_BASIS):
            hn = c2 * hm1 - hm2
            h_ref[nn, 0] = hn
            hm2, hm1 = hm1, hn

    h4, a4 = pl.pallas_call(
        body,
        grid=(nb,),
        in_specs=[pl.BlockSpec((1, 8, 512), lambda i: (i, 0, 0))] * 3,
        out_specs=[pl.BlockSpec((_NUM_BASIS, 1, 8, 512),
                                lambda i: (0, i, 0, 0)),
                   pl.BlockSpec((3, 1, 8, 512), lambda i: (0, i, 0, 0))],
        out_shape=[jax.ShapeDtypeStruct((_NUM_BASIS, nb, 8, 512),
                                        jnp.float32),
                   jax.ShapeDtypeStruct((3, nb, 8, 512), jnp.float32)],
    )(*dr)
    return h4.reshape(_NUM_BASIS, e_pad), a4.reshape(3, e_pad)


def kernel(x, positions, edge_index, embed_node_x, embed_node_z):
    n = positions.shape[0]
    e = edge_index.shape[1]
    pos_flat = positions.T.reshape(-1)
    ei_flat = edge_index.astype(jnp.int32).reshape(-1)
    emb2t = jnp.concatenate([embed_node_x, embed_node_z], axis=1).T
    npad = -n % 3200
    xpad = jnp.pad(x.astype(jnp.int32), (0, npad))
    hx, hz = _tc_node_embed(xpad, emb2t)
    hx = lax.slice(hx, (0, 0), (8, n))
    hz = lax.slice(hz, (0, 0), (8, n))
    e_pad = -(-e // _B) * _B
    d0, d1, d2 = _sc_edge_diff(pos_flat, ei_flat, n, e_pad)
    h16, a3 = _tc_bessel(d0, d1, d2)
    h16 = lax.slice(h16, (0, 0), (_NUM_BASIS, e))
    a3 = lax.slice(a3, (0, 0), (3, e))
    return (hx.T, hz.T, h16.T, a3.T)
